# baseline (speedup 1.0000x reference)
import dataclasses

import jax
import jax.numpy as jnp
from jax import lax
from jax.experimental import pallas as pl
from jax.experimental.pallas import tpu as pltpu
from jax.experimental.pallas import tpu_sc as plsc

N_USERS = 1000000
N_ITEMS = 100000
N_FACTORS = 32
BATCH = 16384

U_SHIFT = 20
V_SHIFT = 17
U_MAIN = (N_USERS // 128) * 128
V_MAIN = (N_ITEMS // 128) * 128
U_TAILBASE = N_FACTORS << U_SHIFT
V_TAILBASE = N_FACTORS << V_SHIFT
TAIL = N_FACTORS * 128

NUM_CORES = 2
NUM_SUBCORES = 16
NUM_WORKERS = NUM_CORES * NUM_SUBCORES
B_PER_W = BATCH // NUM_WORKERS
IDX_CHUNK = 128
CHUNKS_PER_W = B_PER_W // IDX_CHUNK
LANES = 16
VECS_PER_CHUNK = IDX_CHUNK // LANES


def _detile_body(u_t_hbm, v_t_hbm, u_tail_hbm, v_tail_hbm,
                 u_out_hbm, v_out_hbm, sem):
  copies = []
  for k in range(N_FACTORS):
    copies.append(pltpu.make_async_copy(
        u_t_hbm.at[k, pl.ds(0, U_MAIN)],
        u_out_hbm.at[pl.ds(k << U_SHIFT, U_MAIN)], sem))
    copies.append(pltpu.make_async_copy(
        v_t_hbm.at[k, pl.ds(0, V_MAIN)],
        v_out_hbm.at[pl.ds(k << V_SHIFT, V_MAIN)], sem))
  copies.append(pltpu.make_async_copy(
      u_tail_hbm, u_out_hbm.at[pl.ds(U_TAILBASE, TAIL)], sem))
  copies.append(pltpu.make_async_copy(
      v_tail_hbm, v_out_hbm.at[pl.ds(V_TAILBASE, TAIL)], sem))
  for c in copies:
    c.start()
  for c in copies:
    c.wait()


def _detile(u_t, v_t, u_tail, v_tail):
  return pl.pallas_call(
      _detile_body,
      in_specs=[pl.BlockSpec(memory_space=pl.ANY)] * 4,
      out_specs=[pl.BlockSpec(memory_space=pl.ANY)] * 2,
      out_shape=[jax.ShapeDtypeStruct((U_TAILBASE + TAIL,), jnp.float32),
                 jax.ShapeDtypeStruct((V_TAILBASE + TAIL,), jnp.float32)],
      scratch_shapes=[pltpu.SemaphoreType.DMA],
  )(u_t, v_t, u_tail, v_tail)


def _sc_body(users_hbm, items_hbm, u_hbm, v_hbm, out_hbm,
             uidx, vidx, uoff, voff, u_g, v_g, out_v, sem):
  wid = lax.axis_index("s") * NUM_CORES + lax.axis_index("c")

  pltpu.sync_copy(users_hbm.at[pl.ds(wid * CHUNKS_PER_W, CHUNKS_PER_W)], uidx)
  pltpu.sync_copy(items_hbm.at[pl.ds(wid * CHUNKS_PER_W, CHUNKS_PER_W)], vidx)

  for c in range(CHUNKS_PER_W):
    for i in range(VECS_PER_CHUNK):
      s = pl.ds(i * LANES, LANES)
      uq = uidx.at[c][s]
      vq = vidx.at[c][s]
      mu = uq >= U_MAIN
      mv = vq >= V_MAIN
      for k in range(N_FACTORS):
        uoff.at[c, k][s] = jnp.where(
            mu, uq + (U_TAILBASE + (k << 7) - U_MAIN), uq + (k << U_SHIFT))
        voff.at[c, k][s] = jnp.where(
            mv, vq + (V_TAILBASE + (k << 7) - V_MAIN), vq + (k << V_SHIFT))

  copies = []
  for c in range(CHUNKS_PER_W):
    dst = pl.ds(c * IDX_CHUNK, IDX_CHUNK)
    for k in range(N_FACTORS):
      copies.append(pltpu.async_copy(
          u_hbm.at[uoff.at[c, k]], u_g.at[k, dst], sem))
      copies.append(pltpu.async_copy(
          v_hbm.at[voff.at[c, k]], v_g.at[k, dst], sem))
  for cp in copies:
    cp.wait()

  @pl.loop(0, B_PER_W, step=LANES)
  def _(j):
    s = pl.ds(j, LANES)
    acc = u_g.at[0][s] * v_g.at[0][s]
    for k in range(1, N_FACTORS):
      acc = acc + u_g.at[k][s] * v_g.at[k][s]
    out_v[s] = acc

  pltpu.sync_copy(out_v, out_hbm.at[pl.ds(wid * B_PER_W, B_PER_W)])


@jax.jit
def _pmf(users, items, u_t, v_t, u_tail, v_tail):
  u_flat, v_flat = _detile(u_t, v_t, u_tail, v_tail)

  mesh = plsc.VectorSubcoreMesh(
      core_axis_name="c", subcore_axis_name="s",
      num_cores=NUM_CORES, num_subcores=NUM_SUBCORES)
  cp = pltpu.CompilerParams(use_tc_tiling_on_sc=False)
  if "needs_layout_passes" in pltpu.CompilerParams.__dataclass_fields__:
    cp = dataclasses.replace(cp, needs_layout_passes=False)
  run = pl.kernel(
      _sc_body,
      out_type=jax.ShapeDtypeStruct((BATCH,), jnp.float32),
      mesh=mesh,
      compiler_params=cp,
      scratch_types=[
          pltpu.VMEM((CHUNKS_PER_W, IDX_CHUNK), jnp.int32),
          pltpu.VMEM((CHUNKS_PER_W, IDX_CHUNK), jnp.int32),
          pltpu.VMEM((CHUNKS_PER_W, N_FACTORS, IDX_CHUNK), jnp.int32),
          pltpu.VMEM((CHUNKS_PER_W, N_FACTORS, IDX_CHUNK), jnp.int32),
          pltpu.VMEM((N_FACTORS, B_PER_W), jnp.float32),
          pltpu.VMEM((N_FACTORS, B_PER_W), jnp.float32),
          pltpu.VMEM((B_PER_W,), jnp.float32),
          pltpu.SemaphoreType.DMA,
      ],
  )
  return run(users, items, u_flat, v_flat)


def kernel(users_index, items_index, U, V):
  users = users_index.astype(jnp.int32).reshape(BATCH // IDX_CHUNK, IDX_CHUNK)
  items = items_index.astype(jnp.int32).reshape(BATCH // IDX_CHUNK, IDX_CHUNK)
  u_t = U.T
  v_t = V.T
  u_tail = jnp.pad(u_t[:, U_MAIN:], ((0, 0), (0, 128 - (N_USERS - U_MAIN))))
  v_tail = jnp.pad(v_t[:, V_MAIN:], ((0, 0), (0, 128 - (N_ITEMS - V_MAIN))))
  return _pmf(users, items, u_t, v_t,
              u_tail.reshape(TAIL), v_tail.reshape(TAIL))


# speedup vs baseline: 27.2564x; 27.2564x over previous
import dataclasses

import jax
import jax.numpy as jnp
from jax import lax
from jax.experimental import pallas as pl
from jax.experimental.pallas import tpu as pltpu
from jax.experimental.pallas import tpu_sc as plsc

N_USERS = 1000000
N_ITEMS = 100000
N_FACTORS = 32
BATCH = 16384

CHUNK_SHIFT = 17
CHUNK = 1 << CHUNK_SHIFT
U_CHUNKS = -(-N_USERS // CHUNK)
V_CHUNKS = -(-N_ITEMS // CHUNK)
BANDS = N_FACTORS // 8
U_FLAT = BANDS * U_CHUNKS << 20
V_FLAT = BANDS * V_CHUNKS << 20

NUM_CORES = 2
NUM_SUBCORES = 16
NUM_WORKERS = NUM_CORES * NUM_SUBCORES
B_PER_W = BATCH // NUM_WORKERS
IDX_CHUNK = 128
CHUNKS_PER_W = B_PER_W // IDX_CHUNK
LANES = 16
VECS_PER_CHUNK = IDX_CHUNK // LANES


def _detile_body(i_ref, o_ref):
  o_ref[...] = i_ref[...].reshape(8 * CHUNK)


def _detile(table_t, n_chunks):
  return pl.pallas_call(
      _detile_body,
      grid=(BANDS, n_chunks),
      in_specs=[pl.BlockSpec((8, CHUNK), lambda a, c: (a, c))],
      out_specs=pl.BlockSpec((8 * CHUNK,),
                             lambda a, c, _n=n_chunks: (a * _n + c,)),
      out_shape=jax.ShapeDtypeStruct((BANDS * n_chunks * 8 * CHUNK,),
                                     jnp.float32),
  )(table_t)


def _flat_const(k, n_chunks):
  return (((k >> 3) * n_chunks) << 20) + ((k & 7) << CHUNK_SHIFT)


def _sc_body(users_hbm, items_hbm, u_hbm, v_hbm, out_hbm,
             uidx, vidx, uoff, voff, u_g, v_g, out_v, sem):
  wid = lax.axis_index("s") * NUM_CORES + lax.axis_index("c")

  pltpu.sync_copy(users_hbm.at[pl.ds(wid * CHUNKS_PER_W, CHUNKS_PER_W)], uidx)
  pltpu.sync_copy(items_hbm.at[pl.ds(wid * CHUNKS_PER_W, CHUNKS_PER_W)], vidx)

  for c in range(CHUNKS_PER_W):
    for i in range(VECS_PER_CHUNK):
      s = pl.ds(i * LANES, LANES)
      uq = uidx.at[c][s]
      vq = vidx.at[c][s]
      up = ((uq >> CHUNK_SHIFT) << 20) + (uq & (CHUNK - 1))
      for k in range(N_FACTORS):
        uoff.at[c, k][s] = up + _flat_const(k, U_CHUNKS)
        voff.at[c, k][s] = vq + _flat_const(k, V_CHUNKS)

  copies = []
  for c in range(CHUNKS_PER_W):
    dst = pl.ds(c * IDX_CHUNK, IDX_CHUNK)
    for k in range(N_FACTORS):
      copies.append(pltpu.async_copy(
          u_hbm.at[uoff.at[c, k]], u_g.at[k, dst], sem))
      copies.append(pltpu.async_copy(
          v_hbm.at[voff.at[c, k]], v_g.at[k, dst], sem))
  for cp in copies:
    cp.wait()

  @pl.loop(0, B_PER_W, step=LANES)
  def _(j):
    s = pl.ds(j, LANES)
    acc = u_g.at[0][s] * v_g.at[0][s]
    for k in range(1, N_FACTORS):
      acc = acc + u_g.at[k][s] * v_g.at[k][s]
    out_v[s] = acc

  pltpu.sync_copy(out_v, out_hbm.at[pl.ds(wid * B_PER_W, B_PER_W)])


@jax.jit
def _pmf(users, items, u_t, v_t):
  u_flat = _detile(u_t, U_CHUNKS)
  v_flat = _detile(v_t, V_CHUNKS)

  mesh = plsc.VectorSubcoreMesh(
      core_axis_name="c", subcore_axis_name="s",
      num_cores=NUM_CORES, num_subcores=NUM_SUBCORES)
  cp = pltpu.CompilerParams(use_tc_tiling_on_sc=False)
  if "needs_layout_passes" in pltpu.CompilerParams.__dataclass_fields__:
    cp = dataclasses.replace(cp, needs_layout_passes=False)
  run = pl.kernel(
      _sc_body,
      out_type=jax.ShapeDtypeStruct((BATCH,), jnp.float32),
      mesh=mesh,
      compiler_params=cp,
      scratch_types=[
          pltpu.VMEM((CHUNKS_PER_W, IDX_CHUNK), jnp.int32),
          pltpu.VMEM((CHUNKS_PER_W, IDX_CHUNK), jnp.int32),
          pltpu.VMEM((CHUNKS_PER_W, N_FACTORS, IDX_CHUNK), jnp.int32),
          pltpu.VMEM((CHUNKS_PER_W, N_FACTORS, IDX_CHUNK), jnp.int32),
          pltpu.VMEM((N_FACTORS, B_PER_W), jnp.float32),
          pltpu.VMEM((N_FACTORS, B_PER_W), jnp.float32),
          pltpu.VMEM((B_PER_W,), jnp.float32),
          pltpu.SemaphoreType.DMA,
      ],
  )
  return run(users, items, u_flat, v_flat)


def kernel(users_index, items_index, U, V):
  users = users_index.astype(jnp.int32).reshape(BATCH // IDX_CHUNK, IDX_CHUNK)
  items = items_index.astype(jnp.int32).reshape(BATCH // IDX_CHUNK, IDX_CHUNK)
  return _pmf(users, items, U.T, V.T)


# speedup vs baseline: 29.0818x; 1.0670x over previous
import dataclasses

import jax
import jax.numpy as jnp
from jax import lax
from jax.experimental import pallas as pl
from jax.experimental.pallas import tpu as pltpu
from jax.experimental.pallas import tpu_sc as plsc

N_USERS = 1000000
N_ITEMS = 100000
N_FACTORS = 32
BATCH = 16384

CHUNK_SHIFT = 17
CHUNK = 1 << CHUNK_SHIFT
U_CHUNKS = -(-N_USERS // CHUNK)
V_CHUNKS = -(-N_ITEMS // CHUNK)
BANDS = N_FACTORS // 8

NUM_CORES = 2
NUM_SUBCORES = 16
NUM_WORKERS = NUM_CORES * NUM_SUBCORES
B_PER_W = BATCH // NUM_WORKERS
IDX_CHUNK = 128
CHUNKS_PER_W = B_PER_W // IDX_CHUNK
LANES = 16
VECS_PER_CHUNK = IDX_CHUNK // LANES
STAGE = NUM_WORKERS * N_FACTORS * B_PER_W


def _detile_body(i_ref, o_ref):
  o_ref[...] = i_ref[...].reshape(8 * CHUNK)


def _detile(table_t, n_chunks):
  return pl.pallas_call(
      _detile_body,
      grid=(BANDS, n_chunks),
      in_specs=[pl.BlockSpec((8, CHUNK), lambda a, c: (a, c))],
      out_specs=pl.BlockSpec((8 * CHUNK,),
                             lambda a, c, _n=n_chunks: (a * _n + c,)),
      out_shape=jax.ShapeDtypeStruct((BANDS * n_chunks * 8 * CHUNK,),
                                     jnp.float32),
  )(table_t)


def _flat_const(k, n_chunks):
  return (((k >> 3) * n_chunks) << 20) + ((k & 7) << CHUNK_SHIFT)


def _worker_id():
  return lax.axis_index("s") * NUM_CORES + lax.axis_index("c")


def _load_idx(idx_hbm, idx_vmem, wid):
  pltpu.sync_copy(
      idx_hbm.at[pl.ds(wid * CHUNKS_PER_W, CHUNKS_PER_W)], idx_vmem)


def _gather_table(table_hbm, idx, off, g_dst, sem, n_chunks, chunked):
  for c in range(CHUNKS_PER_W):
    for i in range(VECS_PER_CHUNK):
      s = pl.ds(i * LANES, LANES)
      q = idx.at[c][s]
      if chunked:
        q = ((q >> CHUNK_SHIFT) << 20) + (q & (CHUNK - 1))
      for k in range(N_FACTORS):
        off.at[c, k][s] = q + _flat_const(k, n_chunks)
  copies = []
  for c in range(CHUNKS_PER_W):
    for k in range(N_FACTORS):
      copies.append(pltpu.async_copy(
          table_hbm.at[off.at[c, k]], g_dst(k, c), sem))
  return copies


def _sc_v_body(items_hbm, v_hbm, stage_hbm, vidx, voff, v_g, sem):
  wid = _worker_id()
  _load_idx(items_hbm, vidx, wid)
  dst = lambda k, c: v_g.at[pl.ds(k * B_PER_W + c * IDX_CHUNK, IDX_CHUNK)]
  for cp in _gather_table(v_hbm, vidx, voff, dst, sem, V_CHUNKS, False):
    cp.wait()
  pltpu.sync_copy(
      v_g, stage_hbm.at[pl.ds(wid * N_FACTORS * B_PER_W, N_FACTORS * B_PER_W)])


def _sc_u_body(users_hbm, u_hbm, stage_hbm, out_hbm,
               uidx, uoff, u_g, v_g, out_v, sem):
  wid = _worker_id()
  _load_idx(users_hbm, uidx, wid)
  dst = lambda k, c: u_g.at[k, pl.ds(c * IDX_CHUNK, IDX_CHUNK)]
  copies = _gather_table(u_hbm, uidx, uoff, dst, sem, U_CHUNKS, True)
  pltpu.sync_copy(
      stage_hbm.at[pl.ds(wid * N_FACTORS * B_PER_W, N_FACTORS * B_PER_W)],
      v_g)
  for cp in copies:
    cp.wait()

  @pl.loop(0, B_PER_W, step=LANES)
  def _(j):
    s = pl.ds(j, LANES)
    acc = u_g.at[0][s] * v_g[pl.ds(j, LANES)]
    for k in range(1, N_FACTORS):
      acc = acc + u_g.at[k][s] * v_g[pl.ds(k * B_PER_W + j, LANES)]
    out_v[s] = acc

  pltpu.sync_copy(out_v, out_hbm.at[pl.ds(wid * B_PER_W, B_PER_W)])


def _sc_compiler_params():
  cp = pltpu.CompilerParams(use_tc_tiling_on_sc=False)
  if "needs_layout_passes" in pltpu.CompilerParams.__dataclass_fields__:
    cp = dataclasses.replace(cp, needs_layout_passes=False)
  return cp


@jax.jit
def _pmf(users, items, u_t, v_t):
  mesh = plsc.VectorSubcoreMesh(
      core_axis_name="c", subcore_axis_name="s",
      num_cores=NUM_CORES, num_subcores=NUM_SUBCORES)

  v_flat = _detile(v_t, V_CHUNKS)
  stage = pl.kernel(
      _sc_v_body,
      out_type=jax.ShapeDtypeStruct((STAGE,), jnp.float32),
      mesh=mesh,
      compiler_params=_sc_compiler_params(),
      scratch_types=[
          pltpu.VMEM((CHUNKS_PER_W, IDX_CHUNK), jnp.int32),
          pltpu.VMEM((CHUNKS_PER_W, N_FACTORS, IDX_CHUNK), jnp.int32),
          pltpu.VMEM((N_FACTORS * B_PER_W,), jnp.float32),
          pltpu.SemaphoreType.DMA,
      ],
  )(items, v_flat)

  u_flat = _detile(u_t, U_CHUNKS)
  return pl.kernel(
      _sc_u_body,
      out_type=jax.ShapeDtypeStruct((BATCH,), jnp.float32),
      mesh=mesh,
      compiler_params=_sc_compiler_params(),
      scratch_types=[
          pltpu.VMEM((CHUNKS_PER_W, IDX_CHUNK), jnp.int32),
          pltpu.VMEM((CHUNKS_PER_W, N_FACTORS, IDX_CHUNK), jnp.int32),
          pltpu.VMEM((N_FACTORS, B_PER_W), jnp.float32),
          pltpu.VMEM((N_FACTORS * B_PER_W,), jnp.float32),
          pltpu.VMEM((B_PER_W,), jnp.float32),
          pltpu.SemaphoreType.DMA,
      ],
  )(users, u_flat, stage)


def kernel(users_index, items_index, U, V):
  users = users_index.astype(jnp.int32).reshape(BATCH // IDX_CHUNK, IDX_CHUNK)
  items = items_index.astype(jnp.int32).reshape(BATCH // IDX_CHUNK, IDX_CHUNK)
  return _pmf(users, items, U.T, V.T)


# speedup vs baseline: 34.2906x; 1.1791x over previous
import dataclasses

import jax
import jax.numpy as jnp
from jax import lax
from jax.experimental import pallas as pl
from jax.experimental.pallas import tpu as pltpu
from jax.experimental.pallas import tpu_sc as plsc

N_USERS = 1000000
N_ITEMS = 100000
N_FACTORS = 32
BATCH = 16384

CHUNK_SHIFT = 17
CHUNK = 1 << CHUNK_SHIFT
PAIR_SHIFT = CHUNK_SHIFT + 2
U_CHUNKS = -(-N_USERS // CHUNK)
V_CHUNKS = -(-N_ITEMS // CHUNK)
BANDS = N_FACTORS // 8

NUM_CORES = 2
NUM_SUBCORES = 16
NUM_WORKERS = NUM_CORES * NUM_SUBCORES
B_PER_W = BATCH // NUM_WORKERS
IDX_CHUNK = 128
CHUNKS_PER_W = B_PER_W // IDX_CHUNK
LANES = 16
VECS_PER_CHUNK = IDX_CHUNK // LANES
NPAIRS_STAGE = (N_FACTORS // 2) * B_PER_W
STAGE = NUM_WORKERS * NPAIRS_STAGE


def _detile_body(i_ref, o_ref):
  bits = lax.bitcast_convert_type(i_ref[...], jnp.int32)
  b = lax.shift_right_logical(bits + 0x7FFF + ((bits >> 16) & 1), 16)
  p = b.reshape(4, 2, CHUNK)
  o_ref[...] = (p[:, 0, :] | (p[:, 1, :] << 16)).reshape(4 * CHUNK)


def _detile(table_t, n_chunks):
  return pl.pallas_call(
      _detile_body,
      grid=(BANDS, n_chunks),
      in_specs=[pl.BlockSpec((8, CHUNK), lambda a, c: (a, c))],
      out_specs=pl.BlockSpec((4 * CHUNK,),
                             lambda a, c, _n=n_chunks: (a * _n + c,)),
      out_shape=jax.ShapeDtypeStruct((BANDS * n_chunks * 4 * CHUNK,),
                                     jnp.int32),
  )(table_t)


NPAIRS = N_FACTORS // 2


def _flat_const(kp, n_chunks):
  return ((((kp >> 2) * n_chunks) << PAIR_SHIFT)
          + ((kp & 3) << CHUNK_SHIFT))


def _worker_id():
  return lax.axis_index("s") * NUM_CORES + lax.axis_index("c")


def _load_idx(idx_hbm, idx_vmem, wid):
  pltpu.sync_copy(
      idx_hbm.at[pl.ds(wid * CHUNKS_PER_W, CHUNKS_PER_W)], idx_vmem)


def _gather_table(table_hbm, idx, off, g_dst, sem, n_chunks, chunked):
  for c in range(CHUNKS_PER_W):
    for i in range(VECS_PER_CHUNK):
      s = pl.ds(i * LANES, LANES)
      q = idx.at[c][s]
      if chunked:
        q = ((q >> CHUNK_SHIFT) << PAIR_SHIFT) + (q & (CHUNK - 1))
      for kp in range(NPAIRS):
        off.at[c, kp][s] = q + _flat_const(kp, n_chunks)
  copies = []
  for c in range(CHUNKS_PER_W):
    for kp in range(NPAIRS):
      copies.append(pltpu.async_copy(
          table_hbm.at[off.at[c, kp]], g_dst(kp, c), sem))
  return copies


def _half(q, k):
  if k & 1:
    return plsc.bitcast(q & jnp.int32(-65536), jnp.float32)
  return plsc.bitcast(q << 16, jnp.float32)


def _sc_v_body(items_hbm, v_hbm, stage_hbm, vidx, voff, v_g, sem):
  wid = _worker_id()
  _load_idx(items_hbm, vidx, wid)
  dst = lambda kp, c: v_g.at[pl.ds(kp * B_PER_W + c * IDX_CHUNK, IDX_CHUNK)]
  for cp in _gather_table(v_hbm, vidx, voff, dst, sem, V_CHUNKS, False):
    cp.wait()
  pltpu.sync_copy(
      v_g, stage_hbm.at[pl.ds(wid * NPAIRS_STAGE, NPAIRS_STAGE)])


def _sc_u_body(users_hbm, u_hbm, stage_hbm, out_hbm,
               uidx, uoff, u_g, v_g, out_v, sem):
  wid = _worker_id()
  _load_idx(users_hbm, uidx, wid)
  dst = lambda kp, c: u_g.at[kp, pl.ds(c * IDX_CHUNK, IDX_CHUNK)]
  copies = _gather_table(u_hbm, uidx, uoff, dst, sem, U_CHUNKS, True)
  pltpu.sync_copy(
      stage_hbm.at[pl.ds(wid * NPAIRS_STAGE, NPAIRS_STAGE)], v_g)
  for cp in copies:
    cp.wait()

  @pl.loop(0, B_PER_W, step=LANES)
  def _(j):
    acc = jnp.zeros((LANES,), jnp.float32)
    for k in range(N_FACTORS):
      u = _half(u_g.at[k >> 1][pl.ds(j, LANES)], k)
      v = _half(v_g[pl.ds((k >> 1) * B_PER_W + j, LANES)], k)
      acc = acc + u * v
    out_v[pl.ds(j, LANES)] = acc

  pltpu.sync_copy(out_v, out_hbm.at[pl.ds(wid * B_PER_W, B_PER_W)])


def _sc_compiler_params():
  cp = pltpu.CompilerParams(use_tc_tiling_on_sc=False)
  if "needs_layout_passes" in pltpu.CompilerParams.__dataclass_fields__:
    cp = dataclasses.replace(cp, needs_layout_passes=False)
  return cp


@jax.jit
def _pmf(users, items, u_t, v_t):
  mesh = plsc.VectorSubcoreMesh(
      core_axis_name="c", subcore_axis_name="s",
      num_cores=NUM_CORES, num_subcores=NUM_SUBCORES)

  v_flat = _detile(v_t, V_CHUNKS)
  stage = pl.kernel(
      _sc_v_body,
      out_type=jax.ShapeDtypeStruct((STAGE,), jnp.int32),
      mesh=mesh,
      compiler_params=_sc_compiler_params(),
      scratch_types=[
          pltpu.VMEM((CHUNKS_PER_W, IDX_CHUNK), jnp.int32),
          pltpu.VMEM((CHUNKS_PER_W, NPAIRS, IDX_CHUNK), jnp.int32),
          pltpu.VMEM((NPAIRS_STAGE,), jnp.int32),
          pltpu.SemaphoreType.DMA,
      ],
  )(items, v_flat)

  u_flat = _detile(u_t, U_CHUNKS)
  return pl.kernel(
      _sc_u_body,
      out_type=jax.ShapeDtypeStruct((BATCH,), jnp.float32),
      mesh=mesh,
      compiler_params=_sc_compiler_params(),
      scratch_types=[
          pltpu.VMEM((CHUNKS_PER_W, IDX_CHUNK), jnp.int32),
          pltpu.VMEM((CHUNKS_PER_W, NPAIRS, IDX_CHUNK), jnp.int32),
          pltpu.VMEM((NPAIRS, B_PER_W), jnp.int32),
          pltpu.VMEM((NPAIRS_STAGE,), jnp.int32),
          pltpu.VMEM((B_PER_W,), jnp.float32),
          pltpu.SemaphoreType.DMA,
      ],
  )(users, u_flat, stage)


def kernel(users_index, items_index, U, V):
  users = users_index.astype(jnp.int32).reshape(BATCH // IDX_CHUNK, IDX_CHUNK)
  items = items_index.astype(jnp.int32).reshape(BATCH // IDX_CHUNK, IDX_CHUNK)
  return _pmf(users, items, U.T, V.T)


# speedup vs baseline: 37.0734x; 1.0812x over previous
import dataclasses

import jax
import jax.numpy as jnp
from jax import lax
from jax.experimental import pallas as pl
from jax.experimental.pallas import tpu as pltpu
from jax.experimental.pallas import tpu_sc as plsc

N_USERS = 1000000
N_ITEMS = 100000
N_FACTORS = 32
BATCH = 16384

CHUNK_SHIFT = 17
CHUNK = 1 << CHUNK_SHIFT
PAIR_SHIFT = CHUNK_SHIFT + 2
U_CHUNKS = -(-N_USERS // CHUNK)
V_CHUNKS = -(-N_ITEMS // CHUNK)
BANDS = N_FACTORS // 8

NUM_CORES = 2
NUM_SUBCORES = 16
NUM_WORKERS = NUM_CORES * NUM_SUBCORES
B_PER_W = BATCH // NUM_WORKERS
IDX_CHUNK = 128
CHUNKS_PER_W = B_PER_W // IDX_CHUNK
LANES = 16
VECS_PER_CHUNK = IDX_CHUNK // LANES
NPAIRS_STAGE = (N_FACTORS // 2) * B_PER_W
STAGE = NUM_WORKERS * NPAIRS_STAGE


def _detile_body(i_ref, o_ref):
  bits = lax.bitcast_convert_type(i_ref[...], jnp.int32)
  p = (bits + 0x8000).reshape(4, 2, CHUNK)
  o_ref[...] = (lax.shift_right_logical(p[:, 0, :], 16)
                | (p[:, 1, :] & jnp.int32(-65536))).reshape(4 * CHUNK)


def _detile(table_t, n_chunks):
  return pl.pallas_call(
      _detile_body,
      grid=(BANDS, n_chunks),
      in_specs=[pl.BlockSpec((8, CHUNK), lambda a, c: (a, c))],
      out_specs=pl.BlockSpec((4 * CHUNK,),
                             lambda a, c, _n=n_chunks: (a * _n + c,)),
      out_shape=jax.ShapeDtypeStruct((BANDS * n_chunks * 4 * CHUNK,),
                                     jnp.int32),
  )(table_t)


NPAIRS = N_FACTORS // 2


def _flat_const(kp, n_chunks):
  return ((((kp >> 2) * n_chunks) << PAIR_SHIFT)
          + ((kp & 3) << CHUNK_SHIFT))


def _worker_id():
  return lax.axis_index("s") * NUM_CORES + lax.axis_index("c")


def _load_idx(idx_hbm, idx_vmem, wid):
  pltpu.sync_copy(
      idx_hbm.at[pl.ds(wid * CHUNKS_PER_W, CHUNKS_PER_W)], idx_vmem)


def _gather_table(table_hbm, idx, off, g_dst, sem, n_chunks, chunked):
  for c in range(CHUNKS_PER_W):
    for i in range(VECS_PER_CHUNK):
      s = pl.ds(i * LANES, LANES)
      q = idx.at[c][s]
      if chunked:
        q = ((q >> CHUNK_SHIFT) << PAIR_SHIFT) + (q & (CHUNK - 1))
      for kp in range(NPAIRS):
        off.at[c, kp][s] = q + _flat_const(kp, n_chunks)
  copies = []
  for c in range(CHUNKS_PER_W):
    for kp in range(NPAIRS):
      copies.append(pltpu.async_copy(
          table_hbm.at[off.at[c, kp]], g_dst(kp, c), sem))
  return copies


def _half(q, k):
  if k & 1:
    return plsc.bitcast(q & jnp.int32(-65536), jnp.float32)
  return plsc.bitcast(q << 16, jnp.float32)


def _sc_v_body(items_hbm, v_hbm, stage_hbm, vidx, voff, v_g, sem):
  wid = _worker_id()
  _load_idx(items_hbm, vidx, wid)
  dst = lambda kp, c: v_g.at[pl.ds(kp * B_PER_W + c * IDX_CHUNK, IDX_CHUNK)]
  for cp in _gather_table(v_hbm, vidx, voff, dst, sem, V_CHUNKS, False):
    cp.wait()
  pltpu.sync_copy(
      v_g, stage_hbm.at[pl.ds(wid * NPAIRS_STAGE, NPAIRS_STAGE)])


def _sc_u_body(users_hbm, u_hbm, stage_hbm, out_hbm,
               uidx, uoff, u_g, v_g, out_v, sem):
  wid = _worker_id()
  _load_idx(users_hbm, uidx, wid)
  dst = lambda kp, c: u_g.at[kp, pl.ds(c * IDX_CHUNK, IDX_CHUNK)]
  copies = _gather_table(u_hbm, uidx, uoff, dst, sem, U_CHUNKS, True)
  pltpu.sync_copy(
      stage_hbm.at[pl.ds(wid * NPAIRS_STAGE, NPAIRS_STAGE)], v_g)
  for cp in copies:
    cp.wait()

  @pl.loop(0, B_PER_W, step=LANES)
  def _(j):
    acc = jnp.zeros((LANES,), jnp.float32)
    for k in range(N_FACTORS):
      u = _half(u_g.at[k >> 1][pl.ds(j, LANES)], k)
      v = _half(v_g[pl.ds((k >> 1) * B_PER_W + j, LANES)], k)
      acc = acc + u * v
    out_v[pl.ds(j, LANES)] = acc

  pltpu.sync_copy(out_v, out_hbm.at[pl.ds(wid * B_PER_W, B_PER_W)])


def _sc_compiler_params():
  cp = pltpu.CompilerParams(use_tc_tiling_on_sc=False)
  if "needs_layout_passes" in pltpu.CompilerParams.__dataclass_fields__:
    cp = dataclasses.replace(cp, needs_layout_passes=False)
  return cp


@jax.jit
def _pmf(users, items, u_t, v_t):
  mesh = plsc.VectorSubcoreMesh(
      core_axis_name="c", subcore_axis_name="s",
      num_cores=NUM_CORES, num_subcores=NUM_SUBCORES)

  v_flat = _detile(v_t, V_CHUNKS)
  stage = pl.kernel(
      _sc_v_body,
      out_type=jax.ShapeDtypeStruct((STAGE,), jnp.int32),
      mesh=mesh,
      compiler_params=_sc_compiler_params(),
      scratch_types=[
          pltpu.VMEM((CHUNKS_PER_W, IDX_CHUNK), jnp.int32),
          pltpu.VMEM((CHUNKS_PER_W, NPAIRS, IDX_CHUNK), jnp.int32),
          pltpu.VMEM((NPAIRS_STAGE,), jnp.int32),
          pltpu.SemaphoreType.DMA,
      ],
  )(items, v_flat)

  u_flat = _detile(u_t, U_CHUNKS)
  return pl.kernel(
      _sc_u_body,
      out_type=jax.ShapeDtypeStruct((BATCH,), jnp.float32),
      mesh=mesh,
      compiler_params=_sc_compiler_params(),
      scratch_types=[
          pltpu.VMEM((CHUNKS_PER_W, IDX_CHUNK), jnp.int32),
          pltpu.VMEM((CHUNKS_PER_W, NPAIRS, IDX_CHUNK), jnp.int32),
          pltpu.VMEM((NPAIRS, B_PER_W), jnp.int32),
          pltpu.VMEM((NPAIRS_STAGE,), jnp.int32),
          pltpu.VMEM((B_PER_W,), jnp.float32),
          pltpu.SemaphoreType.DMA,
      ],
  )(users, u_flat, stage)


def kernel(users_index, items_index, U, V):
  users = users_index.astype(jnp.int32).reshape(BATCH // IDX_CHUNK, IDX_CHUNK)
  items = items_index.astype(jnp.int32).reshape(BATCH // IDX_CHUNK, IDX_CHUNK)
  return _pmf(users, items, U.T, V.T)


# speedup vs baseline: 37.8481x; 1.0209x over previous
import dataclasses

import jax
import jax.numpy as jnp
from jax import lax
from jax.experimental import pallas as pl
from jax.experimental.pallas import tpu as pltpu
from jax.experimental.pallas import tpu_sc as plsc

N_USERS = 1000000
N_ITEMS = 100000
N_FACTORS = 32
BATCH = 16384

CHUNK_SHIFT = 17
CHUNK = 1 << CHUNK_SHIFT
PAIR_SHIFT = CHUNK_SHIFT + 2
U_CHUNKS = -(-N_USERS // CHUNK)
V_CHUNKS = -(-N_ITEMS // CHUNK)
BANDS = N_FACTORS // 8

NUM_CORES = 2
NUM_SUBCORES = 16
NUM_WORKERS = NUM_CORES * NUM_SUBCORES
B_PER_W = BATCH // NUM_WORKERS
IDX_CHUNK = 128
CHUNKS_PER_W = B_PER_W // IDX_CHUNK
LANES = 16
VECS_PER_CHUNK = IDX_CHUNK // LANES
NPAIRS_STAGE = (N_FACTORS // 2) * B_PER_W
STAGE = NUM_WORKERS * NPAIRS_STAGE


def _detile_body(i_ref, o_ref):
  bits = lax.bitcast_convert_type(i_ref[...], jnp.int32)
  b = bits + 0x8000
  o_ref[...] = (lax.shift_right_logical(b[0:4, :], 16)
                | (b[4:8, :] & jnp.int32(-65536))).reshape(4 * CHUNK)


def _detile(table_t, n_chunks):
  return pl.pallas_call(
      _detile_body,
      grid=(BANDS, n_chunks),
      in_specs=[pl.BlockSpec((8, CHUNK), lambda a, c: (a, c))],
      out_specs=pl.BlockSpec((4 * CHUNK,),
                             lambda a, c, _n=n_chunks: (a * _n + c,)),
      out_shape=jax.ShapeDtypeStruct((BANDS * n_chunks * 4 * CHUNK,),
                                     jnp.int32),
  )(table_t)


NPAIRS = N_FACTORS // 2


def _flat_const(kp, n_chunks):
  return ((((kp >> 2) * n_chunks) << PAIR_SHIFT)
          + ((kp & 3) << CHUNK_SHIFT))


def _worker_id():
  return lax.axis_index("s") * NUM_CORES + lax.axis_index("c")


def _load_idx(idx_hbm, idx_vmem, wid):
  pltpu.sync_copy(
      idx_hbm.at[pl.ds(wid * CHUNKS_PER_W, CHUNKS_PER_W)], idx_vmem)


def _gather_table(table_hbm, idx, off, g_dst, sem, n_chunks, chunked):
  for c in range(CHUNKS_PER_W):
    for i in range(VECS_PER_CHUNK):
      s = pl.ds(i * LANES, LANES)
      q = idx.at[c][s]
      if chunked:
        q = ((q >> CHUNK_SHIFT) << PAIR_SHIFT) + (q & (CHUNK - 1))
      for kp in range(NPAIRS):
        off.at[c, kp][s] = q + _flat_const(kp, n_chunks)
  copies = []
  for c in range(CHUNKS_PER_W):
    for kp in range(NPAIRS):
      copies.append(pltpu.async_copy(
          table_hbm.at[off.at[c, kp]], g_dst(kp, c), sem))
  return copies


def _pair_row(k):
  return ((k >> 3) << 2) + (k & 3)


def _half(q, k):
  if (k >> 2) & 1:
    return plsc.bitcast(q & jnp.int32(-65536), jnp.float32)
  return plsc.bitcast(q << 16, jnp.float32)


def _sc_v_body(items_hbm, v_hbm, stage_hbm, vidx, voff, v_g, sem):
  wid = _worker_id()
  _load_idx(items_hbm, vidx, wid)
  dst = lambda kp, c: v_g.at[pl.ds(kp * B_PER_W + c * IDX_CHUNK, IDX_CHUNK)]
  for cp in _gather_table(v_hbm, vidx, voff, dst, sem, V_CHUNKS, False):
    cp.wait()
  pltpu.sync_copy(
      v_g, stage_hbm.at[pl.ds(wid * NPAIRS_STAGE, NPAIRS_STAGE)])


def _sc_u_body(users_hbm, u_hbm, stage_hbm, out_hbm,
               uidx, uoff, u_g, v_g, out_v, sem):
  wid = _worker_id()
  _load_idx(users_hbm, uidx, wid)
  dst = lambda kp, c: u_g.at[kp, pl.ds(c * IDX_CHUNK, IDX_CHUNK)]
  copies = _gather_table(u_hbm, uidx, uoff, dst, sem, U_CHUNKS, True)
  pltpu.sync_copy(
      stage_hbm.at[pl.ds(wid * NPAIRS_STAGE, NPAIRS_STAGE)], v_g)
  for cp in copies:
    cp.wait()

  @pl.loop(0, B_PER_W, step=LANES)
  def _(j):
    acc = jnp.zeros((LANES,), jnp.float32)
    for k in range(N_FACTORS):
      u = _half(u_g.at[_pair_row(k)][pl.ds(j, LANES)], k)
      v = _half(v_g[pl.ds(_pair_row(k) * B_PER_W + j, LANES)], k)
      acc = acc + u * v
    out_v[pl.ds(j, LANES)] = acc

  pltpu.sync_copy(out_v, out_hbm.at[pl.ds(wid * B_PER_W, B_PER_W)])


def _sc_compiler_params():
  cp = pltpu.CompilerParams(use_tc_tiling_on_sc=False)
  if "needs_layout_passes" in pltpu.CompilerParams.__dataclass_fields__:
    cp = dataclasses.replace(cp, needs_layout_passes=False)
  return cp


@jax.jit
def _pmf(users, items, u_t, v_t):
  mesh = plsc.VectorSubcoreMesh(
      core_axis_name="c", subcore_axis_name="s",
      num_cores=NUM_CORES, num_subcores=NUM_SUBCORES)

  v_flat = _detile(v_t, V_CHUNKS)
  stage = pl.kernel(
      _sc_v_body,
      out_type=jax.ShapeDtypeStruct((STAGE,), jnp.int32),
      mesh=mesh,
      compiler_params=_sc_compiler_params(),
      scratch_types=[
          pltpu.VMEM((CHUNKS_PER_W, IDX_CHUNK), jnp.int32),
          pltpu.VMEM((CHUNKS_PER_W, NPAIRS, IDX_CHUNK), jnp.int32),
          pltpu.VMEM((NPAIRS_STAGE,), jnp.int32),
          pltpu.SemaphoreType.DMA,
      ],
  )(items, v_flat)

  u_flat = _detile(u_t, U_CHUNKS)
  return pl.kernel(
      _sc_u_body,
      out_type=jax.ShapeDtypeStruct((BATCH,), jnp.float32),
      mesh=mesh,
      compiler_params=_sc_compiler_params(),
      scratch_types=[
          pltpu.VMEM((CHUNKS_PER_W, IDX_CHUNK), jnp.int32),
          pltpu.VMEM((CHUNKS_PER_W, NPAIRS, IDX_CHUNK), jnp.int32),
          pltpu.VMEM((NPAIRS, B_PER_W), jnp.int32),
          pltpu.VMEM((NPAIRS_STAGE,), jnp.int32),
          pltpu.VMEM((B_PER_W,), jnp.float32),
          pltpu.SemaphoreType.DMA,
      ],
  )(users, u_flat, stage)


def kernel(users_index, items_index, U, V):
  users = users_index.astype(jnp.int32).reshape(BATCH // IDX_CHUNK, IDX_CHUNK)
  items = items_index.astype(jnp.int32).reshape(BATCH // IDX_CHUNK, IDX_CHUNK)
  return _pmf(users, items, U.T, V.T)
